# Initial kernel scaffold; baseline (speedup 1.0000x reference)
import jax
import jax.numpy as jnp
from jax.experimental import pallas as pl


def kernel(x, edge_index, edge_weight, W_gat, att_src, att_dst, att_edge, W_edge, b_gat, W_ih, W_hh, b_ih, b_hh, ln_g, ln_b, W1, b1, W2, b2):
    raise NotImplementedError("write your pallas kernel here")


# speedup vs baseline: 70.5659x; 70.5659x over previous
import functools

import jax
import jax.numpy as jnp
from jax import lax
from jax.experimental import pallas as pl
from jax.experimental.pallas import tpu as pltpu
from jax.experimental.pallas import tpu_sc as plsc

B = 2
N = 10000
T = 12
F_IN = 8
E = 160000
H = 4
C = 32
HID = 128
HOR = 12
NTOT = B * N
ROWW = HID + 16
NS = 16
STRIPE = N // NS


def _full16(v):
    return jnp.full((16,), v, jnp.int32)


def _proj_body(x_ref, wg_ref, p_ref, vrow_ref, xpe_ref, ad_ref):
    blk = x_ref.shape[0]
    xp = jnp.dot(x_ref[...], wg_ref[...], preferred_element_type=jnp.float32)
    asd = jnp.dot(xp, p_ref[...], preferred_element_type=jnp.float32)
    pad = jnp.zeros((blk, 12), jnp.float32)
    xpe_ref[...] = jnp.concatenate([xp, asd[:, :4], pad], axis=1)
    ad_ref[...] = jnp.concatenate(
        [asd[:, 4:], jnp.broadcast_to(vrow_ref[...], (blk, 12))], axis=1)


def _run_proj(xflat, w_gat, p_comb, vrow):
    rows = T * NTOT
    blk = 1024
    grid = (pl.cdiv(rows, blk),)
    return pl.pallas_call(
        _proj_body,
        grid=grid,
        in_specs=[
            pl.BlockSpec((blk, F_IN), lambda i: (i, 0)),
            pl.BlockSpec((F_IN, HID), lambda i: (0, 0)),
            pl.BlockSpec((HID, 8), lambda i: (0, 0)),
            pl.BlockSpec((1, 12), lambda i: (0, 0)),
        ],
        out_specs=[
            pl.BlockSpec((blk, ROWW), lambda i: (i, 0)),
            pl.BlockSpec((blk, 16), lambda i: (i, 0)),
        ],
        out_shape=[
            jax.ShapeDtypeStruct((rows, ROWW), jnp.float32),
            jax.ShapeDtypeStruct((rows, 16), jnp.float32),
        ],
    )(xflat, w_gat, p_comb, vrow)


ECHUNK = 4000
EPAD = 2 * ECHUNK


def _edge_body(epack_hbm, ew_hbm, off_hbm, xpe_hbm, ad_hbm,
               seq_hbm,
               epk, ewc, exbuf,
               rows0, rows1, adr0, adr1, acc,
               off_vm, gsem0, gsem1, asem0, asem1):
    c = lax.axis_index("c")
    s = lax.axis_index("s")
    rows = (rows0, rows1)
    adr = (adr0, adr1)
    gsem = (gsem0, gsem1)
    asem = (asem0, asem1)

    pltpu.sync_copy(off_hbm, off_vm)
    start = plsc.load_gather(
        off_vm, [jnp.full((16,), s, jnp.int32)])[0]
    end = plsc.load_gather(
        off_vm, [jnp.full((16,), s + 1, jnp.int32)])[0]
    astart = (start // 16) * 16
    nchunks = (end - astart + (ECHUNK - 1)) // ECHUNK

    iot = lax.broadcasted_iota(jnp.int32, (16,), 0)
    iotc = jnp.minimum(iot, 3)
    rowbase = s * STRIPE

    def _issue_gather(g, b, t):
        packed = epk[pl.ds(g * 16, 16)]
        srcv = lax.shift_right_logical(packed, 14)
        dstv = lax.bitwise_and(packed, 16383)
        off = t * NTOT + c * N
        pltpu.async_copy(xpe_hbm.at[srcv + off], rows[b], gsem[b])
        pltpu.async_copy(ad_hbm.at[dstv + off], adr[b], asem[b])

    def _flush(run_dl, regs):
        fbase = run_dl * ROWW
        a = fbase + HID
        acc[pl.ds(a, 16)] = acc[pl.ds(a, 16)] + regs[0]
        for jj in range(HID // 16):
            a = fbase + 16 * jj
            acc[pl.ds(a, 16)] = acc[pl.ds(a, 16)] + regs[1 + jj]

    def _process(g, b, cbase, run_dl, regs):
        pltpu.make_async_copy(
            xpe_hbm.at[pl.ds(0, 16)], rows[b], gsem[b]).wait()
        pltpu.make_async_copy(
            ad_hbm.at[pl.ds(0, 16)], adr[b], asem[b]).wait()
        packed = epk[pl.ds(g * 16, 16)]
        dstv = lax.bitwise_and(packed, 16383)
        eww = ewc[pl.ds(g * 16, 16)]
        pos = (cbase + g * 16) + iot
        valid = jnp.logical_and(pos >= start, pos < end)
        exs = []
        for h in range(H):
            a_s = plsc.load_gather(rows[b], [iot, _full16(HID + h)])
            a_d = plsc.load_gather(adr[b], [iot, _full16(h)])
            vh = plsc.load_gather(adr[b], [iot, _full16(4 + h)])
            al = a_s + a_d + eww * vh
            al = jnp.where(al >= 0.0, al, 0.2 * al)
            ex = jnp.where(valid, jnp.exp(al), 0.0)
            exs.append(ex)
            plsc.store_scatter(exbuf, [iot * 8 + h], ex)
        dl = jnp.clip(dstv - rowbase, 0, STRIPE - 1)
        for r in range(16):
            dlr = dl[r]
            sch = [exs[h][r] for h in range(H)]
            dvec = jnp.where(
                iot < 4, exbuf[pl.ds(8 * r, 16)], 0.0)
            contrib = [dvec] + [
                rows[b][r, pl.ds(16 * jj, 16)] * sch[jj // 2]
                for jj in range(HID // 16)]
            cond = dlr == run_dl

            @pl.when(jnp.logical_not(cond))
            def _():
                _flush(run_dl, regs)
            regs = [jnp.where(cond, regs[k] + contrib[k], contrib[k])
                    for k in range(9)]
            run_dl = jnp.where(cond, run_dl, dlr)
        return run_dl, regs

    def _t_body(t, carry):
        def _zrow(i, cc):
            for j in range(ROWW // 16):
                acc[pl.ds(i * ROWW + 16 * j, 16)] = jnp.zeros(
                    (16,), jnp.float32)
            return cc
        lax.fori_loop(0, STRIPE, _zrow, 0)

        def _chunk(kc, cc):
            run_dl, regs = cc[0], list(cc[1:])
            cbase = astart + kc * ECHUNK
            pltpu.sync_copy(epack_hbm.at[pl.ds(cbase, ECHUNK)], epk)
            pltpu.sync_copy(ew_hbm.at[pl.ds(cbase, ECHUNK)], ewc)
            _issue_gather(0, 0, t)
            _issue_gather(1, 1, t)

            def _pair(i, c2):
                run_dl, regs = c2[0], list(c2[1:])
                for b in (0, 1):
                    g = 2 * i + b
                    run_dl, regs = _process(g, b, cbase, run_dl, regs)

                    @pl.when(g + 2 < ECHUNK // 16)
                    def _():
                        _issue_gather(g + 2, b, t)
                return (run_dl, *regs)
            return lax.fori_loop(0, ECHUNK // 32, _pair, (run_dl, *regs))

        z16 = jnp.zeros((16,), jnp.float32)
        fin = lax.fori_loop(
            0, nchunks, _chunk,
            (jnp.int32(0), z16, z16, z16, z16, z16, z16, z16, z16, z16))
        _flush(fin[0], list(fin[1:]))
        pltpu.sync_copy(
            acc,
            seq_hbm.at[pl.ds((t * NTOT + c * N + rowbase) * ROWW,
                             STRIPE * ROWW)])
        return carry

    lax.fori_loop(0, T, _t_body, 0)


def _run_edges(epack, ew, offs, xpe, ad16):
    mesh = plsc.VectorSubcoreMesh(
        core_axis_name="c", subcore_axis_name="s", num_cores=2,
        num_subcores=NS)
    f = pl.kernel(
        _edge_body,
        out_type=jax.ShapeDtypeStruct((T * NTOT * ROWW,), jnp.float32),
        mesh=mesh,
        scratch_types=[
            pltpu.VMEM((ECHUNK,), jnp.int32),
            pltpu.VMEM((ECHUNK,), jnp.float32),
            pltpu.VMEM((144,), jnp.float32),
            pltpu.VMEM((16, ROWW), jnp.float32),
            pltpu.VMEM((16, ROWW), jnp.float32),
            pltpu.VMEM((16, 16), jnp.float32),
            pltpu.VMEM((16, 16), jnp.float32),
            pltpu.VMEM((STRIPE * ROWW,), jnp.float32),
            pltpu.VMEM((32,), jnp.int32),
            pltpu.SemaphoreType.DMA,
            pltpu.SemaphoreType.DMA,
            pltpu.SemaphoreType.DMA,
            pltpu.SemaphoreType.DMA,
        ],
        compiler_params=pltpu.CompilerParams(
            use_tc_tiling_on_sc=False, needs_layout_passes=False),
    )
    return f(epack, ew, offs, xpe, ad16)


def _rnn_body(seq_ref, wih_ref, whh_ref, bb_ref, bgat_ref, lng_ref, lnb_ref,
              w1_ref, b1_ref, w2_ref, b2_ref, out_ref):
    blk = seq_ref.shape[1]
    h = jnp.zeros((blk, HID), jnp.float32)
    cst = jnp.zeros((blk, HID), jnp.float32)
    for t in range(T):
        raw = seq_ref[t]
        parts = []
        for hh in range(H):
            d = raw[:, HID + hh:HID + hh + 1] + 1e-16
            parts.append(raw[:, C * hh:C * (hh + 1)] / d)
        xt = jnp.concatenate(parts, axis=1) + bgat_ref[...]
        xt = jnp.where(xt > 0.0, xt, jnp.exp(jnp.minimum(xt, 0.0)) - 1.0)
        g = (jnp.dot(xt, wih_ref[...], preferred_element_type=jnp.float32)
             + jnp.dot(h, whh_ref[...], preferred_element_type=jnp.float32)
             + bb_ref[...])
        ig = jax.nn.sigmoid(g[:, :HID])
        fg = jax.nn.sigmoid(g[:, HID:2 * HID])
        gg = jnp.tanh(g[:, 2 * HID:3 * HID])
        og = jax.nn.sigmoid(g[:, 3 * HID:])
        cst = fg * cst + ig * gg
        h = og * jnp.tanh(cst)
    mu = jnp.mean(h, axis=1, keepdims=True)
    var = jnp.mean((h - mu) ** 2, axis=1, keepdims=True)
    z = (h - mu) / jnp.sqrt(var + 1e-5) * lng_ref[...] + lnb_ref[...]
    z = jnp.maximum(
        jnp.dot(z, w1_ref[...], preferred_element_type=jnp.float32)
        + b1_ref[...], 0.0)
    out_ref[...] = (jnp.dot(z, w2_ref[...], preferred_element_type=jnp.float32)
                    + b2_ref[...])


def _run_rnn(seq3, wihT, whhT, bb, bgat, lng, lnb, w1, b1, w2, b2):
    blk = 512
    grid = (pl.cdiv(NTOT, blk),)
    return pl.pallas_call(
        _rnn_body,
        grid=grid,
        in_specs=[
            pl.BlockSpec((T, blk, ROWW), lambda i: (0, i, 0)),
            pl.BlockSpec((HID, 4 * HID), lambda i: (0, 0)),
            pl.BlockSpec((HID, 4 * HID), lambda i: (0, 0)),
            pl.BlockSpec((1, 4 * HID), lambda i: (0, 0)),
            pl.BlockSpec((1, HID), lambda i: (0, 0)),
            pl.BlockSpec((1, HID), lambda i: (0, 0)),
            pl.BlockSpec((1, HID), lambda i: (0, 0)),
            pl.BlockSpec((HID, HID), lambda i: (0, 0)),
            pl.BlockSpec((1, HID), lambda i: (0, 0)),
            pl.BlockSpec((HID, HOR), lambda i: (0, 0)),
            pl.BlockSpec((1, HOR), lambda i: (0, 0)),
        ],
        out_specs=pl.BlockSpec((blk, HOR), lambda i: (i, 0)),
        out_shape=jax.ShapeDtypeStruct((NTOT, HOR), jnp.float32),
    )(seq3, wihT, whhT, bb, bgat, lng, lnb, w1, b1, w2, b2)


@jax.jit
def kernel(x, edge_index, edge_weight, W_gat, att_src, att_dst, att_edge,
           W_edge, b_gat, W_ih, W_hh, b_ih, b_hh, ln_g, ln_b, W1, b1, W2, b2):
    hidx = jnp.arange(HID)
    P_src = jnp.zeros((HID, 8), jnp.float32).at[hidx, hidx // C].set(
        att_src.reshape(-1))
    p_comb = P_src.at[hidx, 4 + hidx // C].set(att_dst.reshape(-1))
    v = (W_edge.reshape(H, C) * att_edge).sum(-1)
    vrow = jnp.zeros((1, 12), jnp.float32).at[0, :H].set(v)
    xflat = jnp.transpose(x, (2, 0, 1, 3)).reshape(T * NTOT, F_IN)
    ei = edge_index.astype(jnp.int32)
    perm = jnp.argsort(ei[1])
    dsts = ei[1, perm]
    epack = jnp.concatenate(
        [ei[0, perm] * 16384 + dsts, jnp.zeros((EPAD,), jnp.int32)])
    ew = jnp.concatenate(
        [edge_weight.astype(jnp.float32)[perm], jnp.zeros((EPAD,))])
    offs = jnp.searchsorted(dsts, jnp.arange(NS + 1) * STRIPE).astype(
        jnp.int32)
    offs = jnp.concatenate(
        [offs, jnp.full((32 - NS - 1,), E, jnp.int32)])

    xpe, ad16 = _run_proj(xflat, W_gat, p_comb, vrow)
    seq = _run_edges(epack, ew, offs, xpe, ad16)
    seq3 = seq.reshape(T, NTOT, ROWW)

    pred = _run_rnn(
        seq3, W_ih.T, W_hh.T, (b_ih + b_hh).reshape(1, 4 * HID),
        b_gat.reshape(1, HID), ln_g.reshape(1, HID), ln_b.reshape(1, HID),
        W1, b1.reshape(1, HID), W2, b2.reshape(1, HOR))
    return pred.reshape(B, N, HOR)


# speedup vs baseline: 70.5671x; 1.0000x over previous
import jax
import jax.numpy as jnp
from jax import lax
from jax.experimental import pallas as pl
from jax.experimental.pallas import tpu as pltpu
from jax.experimental.pallas import tpu_sc as plsc

B = 2
N = 10000
T = 12
F_IN = 8
E = 160000
H = 4
C = 32
HID = 128
HOR = 12
NTOT = B * N
ROWW = HID + 16
NS = 16
STRIPE = N // NS


def _full16(v):
    return jnp.full((16,), v, jnp.int32)


def _proj_body(x_ref, wg_ref, p_ref, vrow_ref, xpe_ref, ad_ref):
    blk = x_ref.shape[0]
    xp = jnp.dot(x_ref[...], wg_ref[...], preferred_element_type=jnp.float32)
    asd = jnp.dot(xp, p_ref[...], preferred_element_type=jnp.float32)
    pad = jnp.zeros((blk, 12), jnp.float32)
    xpe_ref[...] = jnp.concatenate([xp, asd[:, :4], pad], axis=1)
    ad_ref[...] = jnp.concatenate(
        [asd[:, 4:], jnp.broadcast_to(vrow_ref[...], (blk, 12))], axis=1)


def _run_proj(xflat, w_gat, p_comb, vrow):
    rows = T * NTOT
    blk = 1024
    grid = (pl.cdiv(rows, blk),)
    return pl.pallas_call(
        _proj_body,
        grid=grid,
        in_specs=[
            pl.BlockSpec((blk, F_IN), lambda i: (i, 0)),
            pl.BlockSpec((F_IN, HID), lambda i: (0, 0)),
            pl.BlockSpec((HID, 8), lambda i: (0, 0)),
            pl.BlockSpec((1, 12), lambda i: (0, 0)),
        ],
        out_specs=[
            pl.BlockSpec((blk, ROWW), lambda i: (i, 0)),
            pl.BlockSpec((blk, 16), lambda i: (i, 0)),
        ],
        out_shape=[
            jax.ShapeDtypeStruct((rows, ROWW), jnp.float32),
            jax.ShapeDtypeStruct((rows, 16), jnp.float32),
        ],
    )(xflat, w_gat, p_comb, vrow)


ECHUNK = 4000
EPAD = 2 * ECHUNK


def _edge_body(epack_hbm, ew_hbm, off_hbm, xpe_hbm, ad_hbm,
               seq_hbm,
               epk, ewc, exbuf,
               rows0, rows1, adr0, adr1, acc,
               off_vm, gsem0, gsem1, asem0, asem1):
    c = lax.axis_index("c")
    s = lax.axis_index("s")
    rows = (rows0, rows1)
    adr = (adr0, adr1)
    gsem = (gsem0, gsem1)
    asem = (asem0, asem1)

    pltpu.sync_copy(off_hbm, off_vm)
    start = plsc.load_gather(
        off_vm, [jnp.full((16,), s, jnp.int32)])[0]
    end = plsc.load_gather(
        off_vm, [jnp.full((16,), s + 1, jnp.int32)])[0]
    astart = (start // 16) * 16
    nchunks = (end - astart + (ECHUNK - 1)) // ECHUNK

    iot = lax.broadcasted_iota(jnp.int32, (16,), 0)
    rowbase = s * STRIPE

    def _issue_gather(g, b, t):
        packed = epk[pl.ds(g * 16, 16)]
        srcv = lax.shift_right_logical(packed, 14)
        dstv = lax.bitwise_and(packed, 16383)
        off = t * NTOT + c * N
        pltpu.async_copy(xpe_hbm.at[srcv + off], rows[b], gsem[b])
        pltpu.async_copy(ad_hbm.at[dstv + off], adr[b], asem[b])

    def _flush(run_dl, regs):
        fbase = run_dl * ROWW
        a = fbase + HID
        acc[pl.ds(a, 16)] = acc[pl.ds(a, 16)] + regs[0]
        for jj in range(HID // 16):
            a = fbase + 16 * jj
            acc[pl.ds(a, 16)] = acc[pl.ds(a, 16)] + regs[1 + jj]

    def _process(g, b, cbase, run_dl, regs):
        pltpu.make_async_copy(
            xpe_hbm.at[pl.ds(0, 16)], rows[b], gsem[b]).wait()
        pltpu.make_async_copy(
            ad_hbm.at[pl.ds(0, 16)], adr[b], asem[b]).wait()
        packed = epk[pl.ds(g * 16, 16)]
        dstv = lax.bitwise_and(packed, 16383)
        eww = ewc[pl.ds(g * 16, 16)]
        pos = (cbase + g * 16) + iot
        valid = jnp.logical_and(pos >= start, pos < end)
        exs = []
        for h in range(H):
            a_s = plsc.load_gather(rows[b], [iot, _full16(HID + h)])
            a_d = plsc.load_gather(adr[b], [iot, _full16(h)])
            vh = plsc.load_gather(adr[b], [iot, _full16(4 + h)])
            al = a_s + a_d + eww * vh
            al = jnp.where(al >= 0.0, al, 0.2 * al)
            ex = jnp.where(valid, jnp.exp(al), 0.0)
            exs.append(ex)
            plsc.store_scatter(exbuf, [iot * 8 + h], ex)
        dl = jnp.clip(dstv - rowbase, 0, STRIPE - 1)
        for r in range(16):
            dlr = dl[r]
            sch = [exs[h][r] for h in range(H)]
            dvec = jnp.where(
                iot < 4, exbuf[pl.ds(8 * r, 16)], 0.0)
            contrib = [dvec] + [
                rows[b][r, pl.ds(16 * jj, 16)] * sch[jj // 2]
                for jj in range(HID // 16)]
            cond = dlr == run_dl

            @pl.when(jnp.logical_not(cond))
            def _():
                _flush(run_dl, regs)
            regs = [jnp.where(cond, regs[k] + contrib[k], contrib[k])
                    for k in range(9)]
            run_dl = jnp.where(cond, run_dl, dlr)
        return run_dl, regs

    def _t_body(t, carry):
        def _zrow(i, cc):
            for j in range(ROWW // 16):
                acc[pl.ds(i * ROWW + 16 * j, 16)] = jnp.zeros(
                    (16,), jnp.float32)
            return cc
        lax.fori_loop(0, STRIPE, _zrow, 0)

        def _chunk(kc, cc):
            run_dl, regs = cc[0], list(cc[1:])
            cbase = astart + kc * ECHUNK
            pltpu.sync_copy(epack_hbm.at[pl.ds(cbase, ECHUNK)], epk)
            pltpu.sync_copy(ew_hbm.at[pl.ds(cbase, ECHUNK)], ewc)
            _issue_gather(0, 0, t)
            _issue_gather(1, 1, t)

            def _pair(i, c2):
                run_dl, regs = c2[0], list(c2[1:])
                for b in (0, 1):
                    g = 2 * i + b
                    run_dl, regs = _process(g, b, cbase, run_dl, regs)

                    @pl.when(g + 2 < ECHUNK // 16)
                    def _():
                        _issue_gather(g + 2, b, t)
                return (run_dl, *regs)
            return lax.fori_loop(0, ECHUNK // 32, _pair, (run_dl, *regs))

        z16 = jnp.zeros((16,), jnp.float32)
        fin = lax.fori_loop(
            0, nchunks, _chunk,
            (jnp.int32(0), z16, z16, z16, z16, z16, z16, z16, z16, z16))
        _flush(fin[0], list(fin[1:]))
        pltpu.sync_copy(
            acc,
            seq_hbm.at[pl.ds((t * NTOT + c * N + rowbase) * ROWW,
                             STRIPE * ROWW)])
        return carry

    lax.fori_loop(0, T, _t_body, 0)


def _run_edges(epack, ew, offs, xpe, ad16):
    mesh = plsc.VectorSubcoreMesh(
        core_axis_name="c", subcore_axis_name="s", num_cores=2,
        num_subcores=NS)
    f = pl.kernel(
        _edge_body,
        out_type=jax.ShapeDtypeStruct((T * NTOT * ROWW,), jnp.float32),
        mesh=mesh,
        scratch_types=[
            pltpu.VMEM((ECHUNK,), jnp.int32),
            pltpu.VMEM((ECHUNK,), jnp.float32),
            pltpu.VMEM((144,), jnp.float32),
            pltpu.VMEM((16, ROWW), jnp.float32),
            pltpu.VMEM((16, ROWW), jnp.float32),
            pltpu.VMEM((16, 16), jnp.float32),
            pltpu.VMEM((16, 16), jnp.float32),
            pltpu.VMEM((STRIPE * ROWW,), jnp.float32),
            pltpu.VMEM((32,), jnp.int32),
            pltpu.SemaphoreType.DMA,
            pltpu.SemaphoreType.DMA,
            pltpu.SemaphoreType.DMA,
            pltpu.SemaphoreType.DMA,
        ],
        compiler_params=pltpu.CompilerParams(
            use_tc_tiling_on_sc=False, needs_layout_passes=False),
    )
    return f(epack, ew, offs, xpe, ad16)


def _rnn_body(seq_ref, wih_ref, whh_ref, bb_ref, bgat_ref, lng_ref, lnb_ref,
              w1_ref, b1_ref, w2_ref, b2_ref, out_ref):
    blk = seq_ref.shape[1]
    h = jnp.zeros((blk, HID), jnp.float32)
    cst = jnp.zeros((blk, HID), jnp.float32)
    for t in range(T):
        raw = seq_ref[t]
        parts = []
        for hh in range(H):
            d = raw[:, HID + hh:HID + hh + 1] + 1e-16
            parts.append(raw[:, C * hh:C * (hh + 1)] / d)
        xt = jnp.concatenate(parts, axis=1) + bgat_ref[...]
        xt = jnp.where(xt > 0.0, xt, jnp.exp(jnp.minimum(xt, 0.0)) - 1.0)
        g = (jnp.dot(xt, wih_ref[...], preferred_element_type=jnp.float32)
             + jnp.dot(h, whh_ref[...], preferred_element_type=jnp.float32)
             + bb_ref[...])
        ig = jax.nn.sigmoid(g[:, :HID])
        fg = jax.nn.sigmoid(g[:, HID:2 * HID])
        gg = jnp.tanh(g[:, 2 * HID:3 * HID])
        og = jax.nn.sigmoid(g[:, 3 * HID:])
        cst = fg * cst + ig * gg
        h = og * jnp.tanh(cst)
    mu = jnp.mean(h, axis=1, keepdims=True)
    var = jnp.mean((h - mu) ** 2, axis=1, keepdims=True)
    z = (h - mu) / jnp.sqrt(var + 1e-5) * lng_ref[...] + lnb_ref[...]
    z = jnp.maximum(
        jnp.dot(z, w1_ref[...], preferred_element_type=jnp.float32)
        + b1_ref[...], 0.0)
    out_ref[...] = (jnp.dot(z, w2_ref[...], preferred_element_type=jnp.float32)
                    + b2_ref[...])


def _run_rnn(seq3, wihT, whhT, bb, bgat, lng, lnb, w1, b1, w2, b2):
    blk = 512
    grid = (pl.cdiv(NTOT, blk),)
    return pl.pallas_call(
        _rnn_body,
        grid=grid,
        in_specs=[
            pl.BlockSpec((T, blk, ROWW), lambda i: (0, i, 0)),
            pl.BlockSpec((HID, 4 * HID), lambda i: (0, 0)),
            pl.BlockSpec((HID, 4 * HID), lambda i: (0, 0)),
            pl.BlockSpec((1, 4 * HID), lambda i: (0, 0)),
            pl.BlockSpec((1, HID), lambda i: (0, 0)),
            pl.BlockSpec((1, HID), lambda i: (0, 0)),
            pl.BlockSpec((1, HID), lambda i: (0, 0)),
            pl.BlockSpec((HID, HID), lambda i: (0, 0)),
            pl.BlockSpec((1, HID), lambda i: (0, 0)),
            pl.BlockSpec((HID, HOR), lambda i: (0, 0)),
            pl.BlockSpec((1, HOR), lambda i: (0, 0)),
        ],
        out_specs=pl.BlockSpec((blk, HOR), lambda i: (i, 0)),
        out_shape=jax.ShapeDtypeStruct((NTOT, HOR), jnp.float32),
    )(seq3, wihT, whhT, bb, bgat, lng, lnb, w1, b1, w2, b2)


@jax.jit
def kernel(x, edge_index, edge_weight, W_gat, att_src, att_dst, att_edge,
           W_edge, b_gat, W_ih, W_hh, b_ih, b_hh, ln_g, ln_b, W1, b1, W2, b2):
    hidx = jnp.arange(HID)
    P_src = jnp.zeros((HID, 8), jnp.float32).at[hidx, hidx // C].set(
        att_src.reshape(-1))
    p_comb = P_src.at[hidx, 4 + hidx // C].set(att_dst.reshape(-1))
    v = (W_edge.reshape(H, C) * att_edge).sum(-1)
    vrow = jnp.zeros((1, 12), jnp.float32).at[0, :H].set(v)
    xflat = jnp.transpose(x, (2, 0, 1, 3)).reshape(T * NTOT, F_IN)
    ei = edge_index.astype(jnp.int32)
    perm = jnp.argsort(ei[1])
    dsts = ei[1, perm]
    epack = jnp.concatenate(
        [ei[0, perm] * 16384 + dsts, jnp.zeros((EPAD,), jnp.int32)])
    ew = jnp.concatenate(
        [edge_weight.astype(jnp.float32)[perm], jnp.zeros((EPAD,))])
    offs = jnp.searchsorted(dsts, jnp.arange(NS + 1) * STRIPE).astype(
        jnp.int32)
    offs = jnp.concatenate(
        [offs, jnp.full((32 - NS - 1,), E, jnp.int32)])

    xpe, ad16 = _run_proj(xflat, W_gat, p_comb, vrow)
    seq = _run_edges(epack, ew, offs, xpe, ad16)
    seq3 = seq.reshape(T, NTOT, ROWW)

    pred = _run_rnn(
        seq3, W_ih.T, W_hh.T, (b_ih + b_hh).reshape(1, 4 * HID),
        b_gat.reshape(1, HID), ln_g.reshape(1, HID), ln_b.reshape(1, HID),
        W1, b1.reshape(1, HID), W2, b2.reshape(1, HOR))
    return pred.reshape(B, N, HOR)
